# baseline (speedup 1.0000x reference)
import functools

import jax
import jax.numpy as jnp
from jax import lax
from jax.experimental import pallas as pl
from jax.experimental.pallas import tpu as pltpu
from jax.experimental.pallas import tpu_sc as plsc

BATCH = 16384
HIST = 200
TABLE_PAD = 1024

NC = 2
NS = 16
LANES = 16
NW = NC * NS

ROWS_PER_W = BATCH // NW
CHUNK_ROWS = 64
NCHUNKS = ROWS_PER_W // CHUNK_ROWS

WORDS = HIST // 2
WINDOWS = tuple(range(0, WORDS - LANES + 1, LANES)) + (WORDS - LANES,)
TAIL_FRESH = LANES * (len(WINDOWS) - 1) - (WORDS - LANES)


def _sc_kernel(rules_hbm, table_hbm, bias_hbm, out_hbm,
               table_v, bias_v, buf0, buf1, out_v, sem0, sem1):
    wid = lax.axis_index("s") * NC + lax.axis_index("c")
    row_base = wid * ROWS_PER_W

    pltpu.sync_copy(table_hbm, table_v)
    pltpu.sync_copy(bias_hbm, bias_v)

    bufs = (buf0, buf1)
    sems = (sem0, sem1)

    def start(c):
        pltpu.async_copy(
            rules_hbm.at[pl.ds(row_base + c * CHUNK_ROWS, CHUNK_ROWS), :],
            bufs[c % 2], sems[c % 2])

    start(0)

    lane_iota = lax.iota(jnp.int32, LANES)
    tail_mask = lane_iota >= TAIL_FRESH
    last_lane = lane_iota == LANES - 1
    fzero = jnp.zeros((LANES,), jnp.float32)

    for c in range(NCHUNKS):
        buf = bufs[c % 2]
        pltpu.make_async_copy(
            rules_hbm.at[pl.ds(row_base + c * CHUNK_ROWS, CHUNK_ROWS), :],
            buf, sems[c % 2]).wait()
        if c + 1 < NCHUNKS:
            start(c + 1)

        out_base = jnp.full((LANES,), c * CHUNK_ROWS, jnp.int32)

        @plsc.parallel_loop(0, CHUNK_ROWS, 1, unroll=2)
        def row_body(r):
            vals = []
            for w in WINDOWS[:-1]:
                pk = buf[r, pl.ds(w, LANES)]
                ia = pk & 0xFFFF
                ib = lax.shift_right_logical(pk, 16)
                vals.append(plsc.load_gather(table_v, [ia]))
                vals.append(plsc.load_gather(table_v, [ib]))
            pk = buf[r, pl.ds(WINDOWS[-1], LANES)]
            ia = pk & 0xFFFF
            ib = lax.shift_right_logical(pk, 16)
            va = plsc.load_gather(table_v, [ia])
            vb = plsc.load_gather(table_v, [ib])
            vals.append(jnp.where(tail_mask, va, fzero))
            vals.append(jnp.where(tail_mask, vb, fzero))
            while len(vals) > 1:
                nxt = [a + b for a, b in zip(vals[::2], vals[1::2])]
                if len(vals) % 2:
                    nxt.append(vals[-1])
                vals = nxt
            total = plsc.cumsum(vals[0]) + bias_v[...]
            plsc.store_scatter(out_v, [out_base + r], total, mask=last_lane)

    pltpu.sync_copy(out_v, out_hbm.at[pl.ds(row_base, ROWS_PER_W)])


@jax.jit
def _run(rules, table_pad, bias16):
    mesh = plsc.VectorSubcoreMesh(
        core_axis_name="c", subcore_axis_name="s",
        num_cores=NC, num_subcores=NS)
    f = pl.kernel(
        _sc_kernel,
        out_type=jax.ShapeDtypeStruct((BATCH,), jnp.float32),
        mesh=mesh,
        scratch_types=[
            pltpu.VMEM((TABLE_PAD,), jnp.float32),
            pltpu.VMEM((LANES,), jnp.float32),
            pltpu.VMEM((CHUNK_ROWS, WORDS), jnp.int32),
            pltpu.VMEM((CHUNK_ROWS, WORDS), jnp.int32),
            pltpu.VMEM((ROWS_PER_W,), jnp.float32),
            pltpu.SemaphoreType.DMA,
            pltpu.SemaphoreType.DMA,
        ],
        compiler_params=pltpu.CompilerParams(
            needs_layout_passes=False, use_tc_tiling_on_sc=True),
    )
    return f(rules, table_pad, bias16)


def kernel(rules, weight, bias):
    table_pad = jnp.pad(weight.reshape(-1), (0, TABLE_PAD - weight.shape[0]))
    bias16 = jnp.broadcast_to(bias.reshape(1), (LANES,))
    rules_pk = lax.bitcast_convert_type(
        rules.astype(jnp.int16).reshape(BATCH, HIST // 2, 2), jnp.int32)
    out = _run(rules_pk, table_pad, bias16)
    return out.reshape(BATCH, 1)


# speedup vs baseline: 2.8477x; 2.8477x over previous
import jax
import jax.numpy as jnp
from jax import lax
from jax.experimental import pallas as pl
from jax.experimental.pallas import tpu as pltpu
from jax.experimental.pallas import tpu_sc as plsc

BATCH = 16384
HIST = 200
TABLE_PAD = 1024

NC = 2
NS = 16
LANES = 16
NW = NC * NS

ROWS_PER_W = BATCH // NW
CHUNK_ROWS = 64
NCHUNKS = ROWS_PER_W // CHUNK_ROWS

WINDOWS = tuple(range(0, HIST - LANES + 1, LANES)) + (HIST - LANES,)
OVERLAP = LANES * (len(WINDOWS) - 1) - (HIST - LANES)


def _sc_kernel(rules_hbm, table_hbm, bias_hbm, out_hbm,
               table_v, bias_v, buf0, buf1, out_v, sem0, sem1):
    wid = lax.axis_index("s") * NC + lax.axis_index("c")
    row_base = wid * ROWS_PER_W

    pltpu.sync_copy(table_hbm, table_v)
    pltpu.sync_copy(bias_hbm, bias_v)

    bufs = (buf0, buf1)
    sems = (sem0, sem1)

    def start(c):
        pltpu.async_copy(
            rules_hbm.at[pl.ds(row_base + c * CHUNK_ROWS, CHUNK_ROWS), :],
            bufs[c % 2], sems[c % 2])

    start(0)

    lane_iota = lax.iota(jnp.int32, LANES)
    tail_mask = lane_iota >= OVERLAP
    last_lane = lane_iota == LANES - 1
    fzero = jnp.zeros((LANES,), jnp.float32)

    for c in range(NCHUNKS):
        buf = bufs[c % 2]
        pltpu.make_async_copy(
            rules_hbm.at[pl.ds(row_base + c * CHUNK_ROWS, CHUNK_ROWS), :],
            buf, sems[c % 2]).wait()
        if c + 1 < NCHUNKS:
            start(c + 1)

        out_base = jnp.full((LANES,), c * CHUNK_ROWS, jnp.int32)

        @plsc.parallel_loop(0, CHUNK_ROWS, 1, unroll=2)
        def row_body(r):
            vals = []
            for w in WINDOWS[:-1]:
                idx = buf[r, pl.ds(w, LANES)]
                vals.append(plsc.load_gather(table_v, [idx]))
            idx = buf[r, pl.ds(WINDOWS[-1], LANES)]
            v = plsc.load_gather(table_v, [idx])
            vals.append(jnp.where(tail_mask, v, fzero))
            while len(vals) > 1:
                nxt = [a + b for a, b in zip(vals[::2], vals[1::2])]
                if len(vals) % 2:
                    nxt.append(vals[-1])
                vals = nxt
            total = plsc.cumsum(vals[0]) + bias_v[...]
            plsc.store_scatter(out_v, [out_base + r], total, mask=last_lane)

    pltpu.sync_copy(out_v, out_hbm.at[pl.ds(row_base, ROWS_PER_W)])


@jax.jit
def _run(rules, table_pad, bias16):
    mesh = plsc.VectorSubcoreMesh(
        core_axis_name="c", subcore_axis_name="s",
        num_cores=NC, num_subcores=NS)
    f = pl.kernel(
        _sc_kernel,
        out_type=jax.ShapeDtypeStruct((BATCH,), jnp.float32),
        mesh=mesh,
        scratch_types=[
            pltpu.VMEM((TABLE_PAD,), jnp.float32),
            pltpu.VMEM((LANES,), jnp.float32),
            pltpu.VMEM((CHUNK_ROWS, HIST), jnp.int32),
            pltpu.VMEM((CHUNK_ROWS, HIST), jnp.int32),
            pltpu.VMEM((ROWS_PER_W,), jnp.float32),
            pltpu.SemaphoreType.DMA,
            pltpu.SemaphoreType.DMA,
        ],
        compiler_params=pltpu.CompilerParams(
            needs_layout_passes=False, use_tc_tiling_on_sc=True),
    )
    return f(rules, table_pad, bias16)


def kernel(rules, weight, bias):
    table_pad = jnp.pad(weight.reshape(-1), (0, TABLE_PAD - weight.shape[0]))
    bias16 = jnp.broadcast_to(bias.reshape(1), (LANES,))
    out = _run(rules, table_pad, bias16)
    return out.reshape(BATCH, 1)


# speedup vs baseline: 2.9067x; 1.0207x over previous
import jax
import jax.numpy as jnp
from jax import lax
from jax.experimental import pallas as pl
from jax.experimental.pallas import tpu as pltpu
from jax.experimental.pallas import tpu_sc as plsc

BATCH = 16384
HIST = 200
TABLE_PAD = 1024

NC = 2
NS = 16
LANES = 16
NW = NC * NS

ROWS_PER_W = BATCH // NW
CHUNK_ROWS = 128
NCHUNKS = ROWS_PER_W // CHUNK_ROWS

WINDOWS = tuple(range(0, HIST - LANES + 1, LANES)) + (HIST - LANES,)
OVERLAP = LANES * (len(WINDOWS) - 1) - (HIST - LANES)


def _sc_kernel(rules_hbm, table_hbm, bias_hbm, out_hbm,
               table_v, bias_v, buf0, buf1, out_v, sem0, sem1):
    wid = lax.axis_index("s") * NC + lax.axis_index("c")
    row_base = wid * ROWS_PER_W

    pltpu.sync_copy(table_hbm, table_v)
    pltpu.sync_copy(bias_hbm, bias_v)

    bufs = (buf0, buf1)
    sems = (sem0, sem1)

    def start(c):
        pltpu.async_copy(
            rules_hbm.at[pl.ds(row_base + c * CHUNK_ROWS, CHUNK_ROWS), :],
            bufs[c % 2], sems[c % 2])

    start(0)

    lane_iota = lax.iota(jnp.int32, LANES)
    tail_mask = lane_iota >= OVERLAP
    last_lane = lane_iota == LANES - 1
    fzero = jnp.zeros((LANES,), jnp.float32)

    for c in range(NCHUNKS):
        buf = bufs[c % 2]
        pltpu.make_async_copy(
            rules_hbm.at[pl.ds(row_base + c * CHUNK_ROWS, CHUNK_ROWS), :],
            buf, sems[c % 2]).wait()
        if c + 1 < NCHUNKS:
            start(c + 1)

        out_base = jnp.full((LANES,), c * CHUNK_ROWS, jnp.int32)

        @plsc.parallel_loop(0, CHUNK_ROWS, 1, unroll=2)
        def row_body(r):
            vals = []
            for w in WINDOWS[:-1]:
                idx = buf[r, pl.ds(w, LANES)]
                vals.append(plsc.load_gather(table_v, [idx]))
            idx = buf[r, pl.ds(WINDOWS[-1], LANES)]
            v = plsc.load_gather(table_v, [idx])
            vals.append(jnp.where(tail_mask, v, fzero))
            while len(vals) > 1:
                nxt = [a + b for a, b in zip(vals[::2], vals[1::2])]
                if len(vals) % 2:
                    nxt.append(vals[-1])
                vals = nxt
            total = plsc.cumsum(vals[0]) + bias_v[...]
            plsc.store_scatter(out_v, [out_base + r], total, mask=last_lane)

    pltpu.sync_copy(out_v, out_hbm.at[pl.ds(row_base, ROWS_PER_W)])


@jax.jit
def _run(rules, table_pad, bias16):
    mesh = plsc.VectorSubcoreMesh(
        core_axis_name="c", subcore_axis_name="s",
        num_cores=NC, num_subcores=NS)
    f = pl.kernel(
        _sc_kernel,
        out_type=jax.ShapeDtypeStruct((BATCH,), jnp.float32),
        mesh=mesh,
        scratch_types=[
            pltpu.VMEM((TABLE_PAD,), jnp.float32),
            pltpu.VMEM((LANES,), jnp.float32),
            pltpu.VMEM((CHUNK_ROWS, HIST), jnp.int32),
            pltpu.VMEM((CHUNK_ROWS, HIST), jnp.int32),
            pltpu.VMEM((ROWS_PER_W,), jnp.float32),
            pltpu.SemaphoreType.DMA,
            pltpu.SemaphoreType.DMA,
        ],
        compiler_params=pltpu.CompilerParams(
            needs_layout_passes=False, use_tc_tiling_on_sc=True),
    )
    return f(rules, table_pad, bias16)


def kernel(rules, weight, bias):
    table_pad = jnp.pad(weight.reshape(-1), (0, TABLE_PAD - weight.shape[0]))
    bias16 = jnp.broadcast_to(bias.reshape(1), (LANES,))
    out = _run(rules, table_pad, bias16)
    return out.reshape(BATCH, 1)
